# Initial kernel scaffold; baseline (speedup 1.0000x reference)
import jax
import jax.numpy as jnp
from jax.experimental import pallas as pl


def kernel(x, lut):
    raise NotImplementedError("write your pallas kernel here")


# speedup vs baseline: 2.4159x; 2.4159x over previous
import functools
import math

import jax
import jax.numpy as jnp
from jax import lax
from jax.experimental import pallas as pl
from jax.experimental.pallas import tpu as pltpu
from jax.experimental.pallas import tpu_sc as plsc

D_MODEL = 128
SCALE = math.sqrt(float(D_MODEL))
NUM_CORES = 2
NUM_SUBCORES = 16
NW = NUM_CORES * NUM_SUBCORES
CHUNK = 128
LANES = 16


@functools.partial(jax.jit, static_argnames=("n_chunks",))
def _emb_call(idx3, lut, n_chunks):
    B = NW * n_chunks * CHUNK

    mesh = plsc.VectorSubcoreMesh(core_axis_name="c", subcore_axis_name="s")

    @functools.partial(
        pl.kernel,
        out_type=jax.ShapeDtypeStruct((B, D_MODEL), jnp.float32),
        mesh=mesh,
        scratch_types=[
            pltpu.VMEM((n_chunks, CHUNK), jnp.int32),
            pltpu.VMEM((CHUNK, D_MODEL), jnp.float32),
            pltpu.SemaphoreType.DMA,
        ],
    )
    def emb(idx_hbm, lut_hbm, out_hbm, idx_v, buf, sem):
        wid = lax.axis_index("s") * NUM_CORES + lax.axis_index("c")
        pltpu.sync_copy(idx_hbm.at[wid], idx_v)

        def chunk_body(j, carry):
            pltpu.async_copy(lut_hbm.at[idx_v.at[j]], buf, sem).wait()

            def row_body(r, c2):
                for k in range(D_MODEL // LANES):
                    sl = pl.ds(k * LANES, LANES)
                    buf[r, sl] = buf[r, sl] * SCALE
                return c2

            lax.fori_loop(0, CHUNK, row_body, 0)
            out_off = (wid * n_chunks + j) * CHUNK
            pltpu.sync_copy(buf, out_hbm.at[pl.ds(out_off, CHUNK)])
            return carry

        lax.fori_loop(0, n_chunks, chunk_body, 0)

    return emb(idx3, lut)


def kernel(x, lut):
    B = x.size
    n_chunks = B // (NW * CHUNK)
    idx3 = x.reshape(NW, n_chunks, CHUNK).astype(jnp.int32)
    out = _emb_call(idx3, lut, n_chunks)
    return out.reshape(*x.shape, D_MODEL)


# speedup vs baseline: 2.9464x; 1.2196x over previous
import functools
import math

import jax
import jax.numpy as jnp
from jax import lax
from jax.experimental import pallas as pl
from jax.experimental.pallas import tpu as pltpu
from jax.experimental.pallas import tpu_sc as plsc

D_MODEL = 128
SCALE = math.sqrt(float(D_MODEL))
NUM_CORES = 2
NUM_SUBCORES = 16
NW = NUM_CORES * NUM_SUBCORES
CHUNK = 128
LANES = 16
NBUF = 5
PRE = 2


@functools.partial(jax.jit, static_argnames=("n_chunks",))
def _emb_call(idx3, lut, n_chunks):
    B = NW * n_chunks * CHUNK
    assert n_chunks % NBUF == 0

    mesh = plsc.VectorSubcoreMesh(core_axis_name="c", subcore_axis_name="s")

    @functools.partial(
        pl.kernel,
        out_type=jax.ShapeDtypeStruct((B, D_MODEL), jnp.float32),
        mesh=mesh,
        scratch_types=[
            pltpu.VMEM((n_chunks, CHUNK), jnp.int32),
            pltpu.VMEM((NBUF, CHUNK, D_MODEL), jnp.float32),
            pltpu.SemaphoreType.DMA((NBUF,)),
            pltpu.SemaphoreType.DMA((NBUF,)),
        ],
    )
    def emb(idx_hbm, lut_hbm, out_hbm, idx_v, bufs, gsem, ssem):
        wid = lax.axis_index("s") * NUM_CORES + lax.axis_index("c")
        pltpu.sync_copy(idx_hbm.at[wid], idx_v)
        base = wid * n_chunks * CHUNK

        def start_gather(j, b):
            pltpu.make_async_copy(
                lut_hbm.at[idx_v.at[j]], bufs.at[b], gsem.at[b]
            ).start()

        def wait_gather(j, b):
            pltpu.make_async_copy(
                lut_hbm.at[idx_v.at[j]], bufs.at[b], gsem.at[b]
            ).wait()

        def scatter_copy(j, b):
            return pltpu.make_async_copy(
                bufs.at[b], out_hbm.at[pl.ds(base + j * CHUNK, CHUNK)], ssem.at[b]
            )

        for b in range(PRE):
            start_gather(b, b)

        def group_body(g, carry):
            for bs in range(NBUF):
                j = g * NBUF + bs
                wait_gather(j, bs)

                def row_body(r, c2):
                    for k in range(D_MODEL // LANES):
                        sl = pl.ds(k * LANES, LANES)
                        bufs[bs, r, sl] = bufs[bs, r, sl] * SCALE
                    return c2

                lax.fori_loop(0, CHUNK, row_body, 0, unroll=2)

                bn = (bs + PRE) % NBUF
                jn = j + PRE

                @pl.when(jn < n_chunks)
                def _():
                    @pl.when(jn >= NBUF)
                    def _():
                        scatter_copy(jn - NBUF, bn).wait()

                    start_gather(jn, bn)

                scatter_copy(j, bs).start()
            return carry

        lax.fori_loop(0, n_chunks // NBUF, group_body, 0)

        for bs in range(NBUF):
            j = n_chunks - NBUF + bs
            scatter_copy(j, bs).wait()

    return emb(idx3, lut)


def kernel(x, lut):
    B = x.size
    n_chunks = B // (NW * CHUNK)
    idx3 = x.reshape(NW, n_chunks, CHUNK).astype(jnp.int32)
    out = _emb_call(idx3, lut, n_chunks)
    return out.reshape(*x.shape, D_MODEL)


# speedup vs baseline: 5.1493x; 1.7477x over previous
import functools
import math

import jax
import jax.numpy as jnp
from jax import lax
from jax.experimental import pallas as pl
from jax.experimental.pallas import tpu as pltpu
from jax.experimental.pallas import tpu_sc as plsc

D_MODEL = 128
SCALE = math.sqrt(float(D_MODEL))
NUM_CORES = 2
NUM_SUBCORES = 16
NW = NUM_CORES * NUM_SUBCORES
ROWS_PER_CHUNK = 2
LANES = 16
NBUF = 4
PRE = 2


@functools.partial(jax.jit, static_argnames=("n_rows", "seq"))
def _emb_call(idx3, lut, n_rows, seq):
    n_chunks = n_rows // (NW * ROWS_PER_CHUNK)
    cw = seq * ROWS_PER_CHUNK
    assert n_chunks % NBUF == 0

    mesh = plsc.VectorSubcoreMesh(core_axis_name="c", subcore_axis_name="s")

    @functools.partial(
        pl.kernel,
        out_type=jax.ShapeDtypeStruct((n_rows, seq, D_MODEL), jnp.float32),
        mesh=mesh,
        scratch_types=[
            pltpu.VMEM((n_chunks, cw), jnp.int32),
            pltpu.VMEM((NBUF, cw, D_MODEL), jnp.float32),
            pltpu.SemaphoreType.DMA((NBUF,)),
            pltpu.SemaphoreType.DMA((NBUF,)),
        ],
    )
    def emb(idx_hbm, lut_hbm, out_hbm, idx_v, bufs, gsem, ssem):
        wid = lax.axis_index("s") * NUM_CORES + lax.axis_index("c")
        pltpu.sync_copy(idx_hbm.at[wid], idx_v)
        row_base = wid * n_chunks * ROWS_PER_CHUNK

        def start_gather(j, b):
            pltpu.make_async_copy(
                lut_hbm.at[idx_v.at[j]], bufs.at[b], gsem.at[b]
            ).start()

        def wait_gather(j, b):
            pltpu.make_async_copy(
                lut_hbm.at[idx_v.at[j]], bufs.at[b], gsem.at[b]
            ).wait()

        def scatter_copies(j, b):
            row0 = row_base + j * ROWS_PER_CHUNK
            return [
                pltpu.make_async_copy(
                    bufs.at[b, pl.ds(r * seq, seq)], out_hbm.at[row0 + r], ssem.at[b]
                )
                for r in range(ROWS_PER_CHUNK)
            ]

        for b in range(PRE):
            start_gather(b, b)

        def group_body(g, carry):
            for bs in range(NBUF):
                j = g * NBUF + bs
                wait_gather(j, bs)

                def row_body(r, c2):
                    for k in range(D_MODEL // LANES):
                        sl = pl.ds(k * LANES, LANES)
                        bufs[bs, r, sl] = bufs[bs, r, sl] * SCALE
                    return c2

                lax.fori_loop(0, cw, row_body, 0, unroll=2)

                bn = (bs + PRE) % NBUF
                jn = j + PRE

                @pl.when(jn < n_chunks)
                def _():
                    @pl.when(jn >= NBUF)
                    def _():
                        for c in scatter_copies(jn - NBUF, bn):
                            c.wait()

                    start_gather(jn, bn)

                for c in scatter_copies(j, bs):
                    c.start()
            return carry

        lax.fori_loop(0, n_chunks // NBUF, group_body, 0)

        for bs in range(NBUF):
            j = n_chunks - NBUF + bs
            for c in scatter_copies(j, bs):
                c.wait()

    return emb(idx3, lut)


def kernel(x, lut):
    n_rows, seq = x.shape
    n_chunks = n_rows // (NW * ROWS_PER_CHUNK)
    idx3 = x.reshape(NW, n_chunks, ROWS_PER_CHUNK * seq).astype(jnp.int32)
    return _emb_call(idx3, lut, n_rows, seq)
